# baseline (speedup 1.0000x reference)
import functools

import jax
import jax.numpy as jnp
from jax import lax
from jax.experimental import pallas as pl
from jax.experimental.pallas import tpu as pltpu
from jax.experimental.pallas import tpu_sc as plsc

D = 128
SEQ = 200
VOCAB = 256
CH = 64
NBUF = 4
LANES = 16
POS_ROWS = 256


def _splat(vec, t):
    return lax.gather(
        vec, jnp.full((LANES, 1), t, jnp.int32),
        lax.GatherDimensionNumbers(
            offset_dims=(), collapsed_slice_dims=(0,), start_index_map=(0,)),
        (1,), mode=lax.GatherScatterMode.PROMISE_IN_BOUNDS)


def _embed_kernel(T, NC, NS):
    NW = NC * NS
    tok_w = T // NW
    nchunk = tok_w // CH
    ngroup = nchunk // NBUF
    mesh = plsc.VectorSubcoreMesh(core_axis_name="c", subcore_axis_name="s")

    @functools.partial(
        pl.kernel,
        mesh=mesh,
        out_type=jax.ShapeDtypeStruct((T, D), jnp.float32),
        compiler_params=pltpu.CompilerParams(needs_layout_passes=False),
        scratch_types=[
            pltpu.VMEM((nchunk * CH // 128, 128), jnp.int32),
            pltpu.VMEM((POS_ROWS, D), jnp.float32),
            pltpu.VMEM((VOCAB * D,), jnp.float32),
            pltpu.VMEM((NBUF, CH, D), jnp.float32),
        ] + [pltpu.SemaphoreType.DMA] * NBUF,
    )
    def k(idx_hbm, pos_hbm, table_hbm, out_hbm,
          idx_v, pos_v, table_v, rows_v, *osem):
        c = lax.axis_index("c")
        s = lax.axis_index("s")
        wid = s * NC + c
        base = wid * tok_w

        nrow = nchunk * CH // 128
        pltpu.sync_copy(idx_hbm.at[pl.ds(wid * nrow, nrow)], idx_v)
        pltpu.sync_copy(pos_hbm, pos_v)
        pltpu.sync_copy(table_hbm, table_v)

        def group_body(g, carry):
            for b in range(NBUF):
                ci = g * NBUF + b

                def wait_out():
                    pltpu.make_async_copy(
                        rows_v.at[b], out_hbm.at[pl.ds(0, CH)],
                        osem[b]).wait()
                pl.when(g >= 1)(wait_out)

                p0 = lax.rem(ci * CH, SEQ)

                irow = 2 * g + b // 2
                ioff = (b % 2) * CH

                def block_body(tb, bc):
                    idx_vec = idx_v[irow, pl.ds(ioff + tb * LANES, LANES)]
                    for t in range(LANES):
                        rowoff = _splat(idx_vec, t) * D
                        pr = p0 + tb * LANES + t
                        for j in range(D // LANES):
                            sl = pl.ds(j * LANES, LANES)
                            col = lax.iota(jnp.int32, LANES) + j * LANES
                            val = plsc.load_gather(table_v, [rowoff + col])
                            rows_v[b, tb * LANES + t, sl] = val + pos_v[pr, sl]
                    return bc

                lax.fori_loop(0, CH // LANES, block_body, 0)

                pltpu.async_copy(
                    rows_v.at[b], out_hbm.at[pl.ds(base + ci * CH, CH)],
                    osem[b])
            return carry

        lax.fori_loop(0, ngroup, group_body, 0)

        for b in range(NBUF):
            pltpu.make_async_copy(
                rows_v.at[b], out_hbm.at[pl.ds(0, CH)], osem[b]).wait()

    return k


def kernel(code_bytes, embedding, positional):
    batch, seq = code_bytes.shape
    T = batch * seq
    idx2d = code_bytes.reshape(T // 128, 128).astype(jnp.int32)
    pos = positional[0, :seq, :]
    pos2 = jnp.concatenate([pos, pos[:POS_ROWS - seq]], axis=0)
    info = plsc.get_sparse_core_info()
    out_flat = _embed_kernel(T, info.num_cores, info.num_subcores)(
        idx2d, pos2, embedding.reshape(-1))
    return out_flat.reshape(batch, seq, D)


# speedup vs baseline: 3.3600x; 3.3600x over previous
import functools

import jax
import jax.numpy as jnp
from jax import lax
from jax.experimental import pallas as pl
from jax.experimental.pallas import tpu as pltpu
from jax.experimental.pallas import tpu_sc as plsc

D = 128
SEQ = 200
CH = 80
NBUF = 5
MAXP0 = max((CH * b) % SEQ for b in range(NBUF))
POS_ROWS = MAXP0 + CH
LANES = 16


def _embed_kernel(T, NC, NS):
    NW = NC * NS
    tok_w = T // NW
    nchunk = tok_w // CH
    ngroup = nchunk // NBUF
    mesh = plsc.VectorSubcoreMesh(core_axis_name="c", subcore_axis_name="s")

    @functools.partial(
        pl.kernel,
        mesh=mesh,
        out_type=jax.ShapeDtypeStruct((T, D), jnp.float32),
        scratch_types=[
            pltpu.VMEM((nchunk, CH), jnp.int32),
            pltpu.VMEM((POS_ROWS, D), jnp.float32),
            pltpu.VMEM((NBUF, CH, D), jnp.float32),
        ] + [pltpu.SemaphoreType.DMA] * (2 * NBUF),
    )
    def k(idx_hbm, pos_hbm, table_hbm, out_hbm, idx_v, pos_v, rows_v, *sems):
        gsem = sems[:NBUF]
        osem = sems[NBUF:]
        c = lax.axis_index("c")
        s = lax.axis_index("s")
        wid = s * NC + c
        base = wid * tok_w

        PF = 3
        pltpu.sync_copy(idx_hbm.at[pl.ds(wid * nchunk, nchunk)], idx_v)
        pltpu.sync_copy(pos_hbm, pos_v)
        woff = jnp.broadcast_to(wid * 256, (LANES,)).astype(jnp.int32)

        def ixf(r, rc):
            for j5 in range(CH // LANES):
                sl5 = pl.ds(j5 * LANES, LANES)
                idx_v[r, sl5] = idx_v[r, sl5] + woff
            return rc

        lax.fori_loop(0, nchunk, ixf, 0)
        for j in range(PF):
            pltpu.async_copy(table_hbm.at[idx_v.at[j]], rows_v.at[j], gsem[j])

        def group_body(g, carry):
            for b in range(NBUF):
                ci = g * NBUF + b
                pslot = (b + PF) % NBUF

                def wait_out():
                    pltpu.make_async_copy(
                        rows_v.at[pslot], out_hbm.at[pl.ds(0, CH)],
                        osem[pslot]).wait()
                if b >= NBUF - PF:
                    wait_out()
                else:
                    pl.when(g >= 1)(wait_out)

                def issue_gather():
                    pltpu.async_copy(
                        table_hbm.at[idx_v.at[ci + PF]], rows_v.at[pslot],
                        gsem[pslot])
                pl.when(ci + PF < nchunk)(issue_gather)

                pltpu.make_async_copy(
                    table_hbm.at[idx_v.at[0]], rows_v.at[b], gsem[b]).wait()

                p0 = (CH * b) % SEQ

                def row_body(r, rc):
                    for u in range(2):
                        rr = r * 2 + u
                        for j in range(D // LANES):
                            sl = pl.ds(j * LANES, LANES)
                            plsc.addupdate(rows_v.at[b, rr, sl],
                                           pos_v[p0 + rr, sl])
                    return rc

                lax.fori_loop(0, CH // 2, row_body, 0)

                pltpu.async_copy(
                    rows_v.at[b], out_hbm.at[pl.ds(base + ci * CH, CH)],
                    osem[b])
            return carry

        lax.fori_loop(0, ngroup, group_body, 0)

        for ci in range(nchunk - (NBUF - PF), nchunk):
            pltpu.make_async_copy(
                rows_v.at[ci % NBUF], out_hbm.at[pl.ds(0, CH)],
                osem[ci % NBUF]).wait()

    return k


def kernel(code_bytes, embedding, positional):
    batch, seq = code_bytes.shape
    T = batch * seq
    idx2d = code_bytes.reshape(T // CH, CH).astype(jnp.int32)
    pos = positional[0, :seq, :]
    pos2 = jnp.concatenate([pos, pos[:POS_ROWS - seq]], axis=0)
    info = plsc.get_sparse_core_info()
    out_flat = _embed_kernel(T, info.num_cores, info.num_subcores)(
        idx2d, pos2, jnp.tile(embedding, (info.num_cores * info.num_subcores, 1)))
    return out_flat.reshape(batch, seq, D)


# speedup vs baseline: 3.9760x; 1.1834x over previous
import functools

import jax
import jax.numpy as jnp
from jax import lax
from jax.experimental import pallas as pl
from jax.experimental.pallas import tpu as pltpu
from jax.experimental.pallas import tpu_sc as plsc

D = 128
SEQ = 200
VOCAB = 256
CH = 80
NBUF = 5
MAXP0 = max((CH * b) % SEQ for b in range(NBUF))
POS_ROWS = MAXP0 + CH
LANES = 16
PF = 3


def _embed_kernel(T, NC, NS):
    NW = NC * NS
    tok_w = T // NW
    nchunk = tok_w // CH
    ngroup = nchunk // NBUF
    mesh = plsc.VectorSubcoreMesh(core_axis_name="c", subcore_axis_name="s")

    @functools.partial(
        pl.kernel,
        mesh=mesh,
        out_type=jax.ShapeDtypeStruct((T, D), jnp.float32),
        scratch_types=[
            pltpu.VMEM((NBUF, CH), jnp.int32),
            pltpu.VMEM((POS_ROWS, D), jnp.float32),
            pltpu.VMEM((NBUF, CH, D), jnp.float32),
            pltpu.VMEM_SHARED((NS * VOCAB, D), jnp.float32),
        ] + [pltpu.SemaphoreType.DMA] * (3 * NBUF),
    )
    def k(idx_hbm, pos_hbm, table_hbm, out_hbm,
          idxb_v, pos_v, rows_v, tab_sh, *sems):
        gsem = sems[:NBUF]
        osem = sems[NBUF:2 * NBUF]
        isem = sems[2 * NBUF:]
        c = lax.axis_index("c")
        s = lax.axis_index("s")
        wid = s * NC + c
        base = wid * tok_w

        pltpu.sync_copy(pos_hbm, pos_v)
        pltpu.sync_copy(table_hbm, tab_sh.at[pl.ds(s * VOCAB, VOCAB)])
        woff = jnp.broadcast_to(s * VOCAB, (LANES,)).astype(jnp.int32)

        def idx_load(cx, slot):
            pltpu.async_copy(idx_hbm.at[pl.ds(base + cx * CH, CH)],
                             idxb_v.at[slot], isem[slot])

        def idx_wait_and_gather(slot):
            pltpu.make_async_copy(idx_hbm.at[pl.ds(0, CH)],
                                  idxb_v.at[slot], isem[slot]).wait()
            for j5 in range(CH // LANES):
                sl5 = pl.ds(j5 * LANES, LANES)
                idxb_v[slot, sl5] = idxb_v[slot, sl5] + woff
            pltpu.async_copy(tab_sh.at[idxb_v.at[slot]], rows_v.at[slot],
                             gsem[slot])

        for j in range(PF + 1):
            idx_load(j, j)
        for j in range(PF):
            idx_wait_and_gather(j)

        def group_body(g, carry):
            for b in range(NBUF):
                ci = g * NBUF + b
                pslot = (b + PF) % NBUF

                def wait_out():
                    pltpu.make_async_copy(
                        rows_v.at[pslot], out_hbm.at[pl.ds(0, CH)],
                        osem[pslot]).wait()
                if b >= NBUF - PF:
                    wait_out()
                else:
                    pl.when(g >= 1)(wait_out)

                pl.when(ci + PF < nchunk)(
                    functools.partial(idx_wait_and_gather, pslot))

                nslot = (pslot + 1) % NBUF
                pl.when(ci + PF + 1 < nchunk)(
                    functools.partial(idx_load, ci + PF + 1, nslot))

                pltpu.make_async_copy(
                    tab_sh.at[idxb_v.at[b]], rows_v.at[b], gsem[b]).wait()

                p0 = (CH * b) % SEQ

                def row_body(r, rc):
                    for u in range(2):
                        rr = r * 2 + u
                        for j in range(D // LANES):
                            sl = pl.ds(j * LANES, LANES)
                            plsc.addupdate(rows_v.at[b, rr, sl],
                                           pos_v[p0 + rr, sl])
                    return rc

                lax.fori_loop(0, CH // 2, row_body, 0)

                pltpu.async_copy(
                    rows_v.at[b], out_hbm.at[pl.ds(base + ci * CH, CH)],
                    osem[b])
            return carry

        lax.fori_loop(0, ngroup, group_body, 0)

        for ci in range(nchunk - (NBUF - PF), nchunk):
            pltpu.make_async_copy(
                rows_v.at[ci % NBUF], out_hbm.at[pl.ds(0, CH)],
                osem[ci % NBUF]).wait()

    return k


def kernel(code_bytes, embedding, positional):
    batch, seq = code_bytes.shape
    T = batch * seq
    idx_flat = code_bytes.reshape(-1).astype(jnp.int32)
    pos = positional[0, :seq, :]
    pos2 = jnp.concatenate([pos, pos[:POS_ROWS - seq]], axis=0)
    info = plsc.get_sparse_core_info()
    out_flat = _embed_kernel(T, info.num_cores, info.num_subcores)(
        idx_flat, pos2, embedding)
    return out_flat.reshape(batch, seq, D)
